# baseline (speedup 1.0000x reference)
import jax
import jax.numpy as jnp
from jax import lax
from jax.experimental import pallas as pl
from jax.experimental.pallas import tpu as pltpu
from jax.experimental.pallas import tpu_sc as plsc

N_NODES = 10000
D_IN = 128
HID = 256
OUT = 128

NC = 2
NS = 16
LN = 128
DUMMY = N_NODES
ACC_ROWS = 10240
ZCH = ACC_ROWS // NS // LN
NGT0 = 96
NGT1 = 64
NGTP = NGT0 + NGT1
IDX_ROWS = NS * NGTP + 16
EDGES_PAD = NS * NGTP * LN
NGH0 = NGT0 // 2


def _sc_mesh():
    return plsc.VectorSubcoreMesh(
        core_axis_name="c", subcore_axis_name="s", num_cores=NC, num_subcores=NS)


def _deg_body(dst_hbm, ones_hbm, zeros_hbm, out_hbm, didx, ones_v, acc,
              ssem):
    c = lax.axis_index("c")
    s = lax.axis_index("s")
    pltpu.sync_copy(zeros_hbm, acc.at[pl.ds(s * 640, 640)])
    pltpu.sync_copy(ones_hbm, ones_v)
    row0 = s * NGTP + c * NGT0
    pltpu.sync_copy(dst_hbm.at[pl.ds(row0, NGT0)], didx)
    plsc.subcore_barrier()
    K = 8

    @pl.loop(0, NGT0 // K - c * ((NGT0 - NGT1) // K))
    def _(g):
        cps = [
            pltpu.async_copy(ones_v, acc.at[didx.at[g * K + j]], ssem, add=True)
            for j in range(K)
        ]
        for cp in cps:
            cp.wait()

    plsc.subcore_barrier()
    pltpu.sync_copy(acc.at[pl.ds(s * 640, 640)],
                    out_hbm.at[c, pl.ds(s * 640, 640)])


def _deg_kernel(dst2d, ones16, zeros16):
    k = pl.kernel(
        _deg_body,
        out_type=jax.ShapeDtypeStruct((NC, ACC_ROWS, 16), jnp.float32),
        mesh=_sc_mesh(),
        compiler_params=pltpu.CompilerParams(use_tc_tiling_on_sc=False),
        scratch_types=[
            pltpu.VMEM((NGT0, LN), jnp.int32),
            pltpu.VMEM((LN, 16), jnp.float32),
            pltpu.VMEM_SHARED((ACC_ROWS, 16), jnp.float32),
            pltpu.SemaphoreType.DMA,
        ],
    )
    return k(dst2d, ones16, zeros16)


def _edge_body(y_hbm, src_hbm, dst_hbm, zeros_hbm, out_hbm,
               sidx, didx, buf0, buf1, acc, gsem, ssem):
    c = lax.axis_index("c")
    s = lax.axis_index("s")
    pltpu.sync_copy(zeros_hbm, acc.at[pl.ds(s * 640, 640)])
    row0 = s * NGTP + c * NGT0
    ngh = NGH0 - c * ((NGT0 - NGT1) // 2)
    pairs = ngh // 2
    pltpu.sync_copy(src_hbm.at[pl.ds(row0, NGH0)], sidx)
    pltpu.sync_copy(dst_hbm.at[pl.ds(row0, NGH0)], didx)
    plsc.subcore_barrier()

    def fire_gather(k, buf):
        pltpu.async_copy(y_hbm.at[sidx.at[k]], buf, gsem)

    def fire_scatter(k, buf):
        pltpu.async_copy(buf, acc.at[didx.at[k]], ssem, add=True)

    def drain(sem, buf):
        pltpu.make_async_copy(y_hbm.at[pl.ds(0, LN)], buf, sem).wait()

    for h in range(2):
        if h:
            pltpu.sync_copy(src_hbm.at[pl.ds(row0 + ngh, NGH0)], sidx)
            pltpu.sync_copy(dst_hbm.at[pl.ds(row0 + ngh, NGH0)], didx)
        fire_gather(0, buf0)

        @pl.loop(0, pairs)
        def _(t):
            k0 = 2 * t
            k1 = k0 + 1
            fire_gather(k1, buf1)
            drain(gsem, buf0)
            fire_scatter(k0, buf0)
            drain(ssem, buf0)

            @pl.when(t < pairs - 1)
            def _():
                fire_gather(k0 + 2, buf0)

            drain(gsem, buf1)
            fire_scatter(k1, buf1)
            drain(ssem, buf1)

    plsc.subcore_barrier()
    pltpu.sync_copy(acc.at[pl.ds(s * 640, 640)],
                    out_hbm.at[c, pl.ds(s * 640, 640)])


def _edge_pass(y, src2d, dst2d, zeros128):
    k = pl.kernel(
        _edge_body,
        out_type=jax.ShapeDtypeStruct((NC, ACC_ROWS, 128), jnp.float32),
        mesh=_sc_mesh(),
        compiler_params=pltpu.CompilerParams(use_tc_tiling_on_sc=False),
        scratch_types=[
            pltpu.VMEM((NGH0, LN), jnp.int32),
            pltpu.VMEM((NGH0, LN), jnp.int32),
            pltpu.VMEM((LN, 128), jnp.float32),
            pltpu.VMEM((LN, 128), jnp.float32),
            pltpu.VMEM_SHARED((ACC_ROWS, 128), jnp.float32),
            pltpu.SemaphoreType.DMA,
            pltpu.SemaphoreType.DMA,
        ],
    )
    return k(y, src2d, dst2d, zeros128)[:, :N_NODES]


R = 1000
GRID = N_NODES // R


def _dinv_of(deg_ref):
    deg = deg_ref[0, :, 0:1] + deg_ref[1, :, 0:1] + 1.0
    return lax.rsqrt(deg)


def _t1_body(x_ref, deg_ref, w1_ref, wu1_ref, bu1_ref, wu2_ref, bu2_ref,
             y1_ref, hnode_ref, gsum_ref):
    i = pl.program_id(0)
    x = x_ref[...]
    dinv = _dinv_of(deg_ref)
    xw = jnp.dot(x, w1_ref[...], preferred_element_type=jnp.float32)
    y1 = xw * dinv
    y1_ref[0] = y1[:, :128]
    y1_ref[1] = y1[:, 128:]
    hu = jnp.maximum(
        jnp.dot(x, wu1_ref[...], preferred_element_type=jnp.float32) + bu1_ref[...],
        0.0)
    hnode = jnp.dot(hu, wu2_ref[...], preferred_element_type=jnp.float32) + bu2_ref[...]
    hnode_ref[...] = hnode
    part = jnp.sum(hnode, axis=0, keepdims=True)

    @pl.when(i == 0)
    def _():
        gsum_ref[...] = part

    @pl.when(i != 0)
    def _():
        gsum_ref[...] += part


def _t2_body(y1_ref, a1a_ref, a1b_ref, deg_ref, b1_ref, w2_ref, y2_ref):
    dinv = _dinv_of(deg_ref)
    ha = a1a_ref[0] + a1a_ref[1] + y1_ref[0]
    hb = a1b_ref[0] + a1b_ref[1] + y1_ref[1]
    h = jnp.concatenate([ha, hb], axis=1) * dinv + b1_ref[...]
    h = jnp.maximum(h, 0.0)
    y2_ref[...] = jnp.dot(h, w2_ref[...], preferred_element_type=jnp.float32) * dinv


def _t3_body(y2_ref, a2_ref, deg_ref, b2_ref, out_ref):
    dinv = _dinv_of(deg_ref)
    out = (a2_ref[0] + a2_ref[1] + y2_ref[...]) * dinv + b2_ref[...]
    out_ref[...] = jnp.maximum(out, 0.0)


def _full(shape):
    return pl.BlockSpec(shape, lambda i: tuple(0 for _ in shape))


def _rows(shape3=None):
    if shape3 is None:
        return pl.BlockSpec((R, 128), lambda i: (i, 0))
    return pl.BlockSpec(shape3, lambda i: (0, i, 0))


def kernel(x, edge_index, W1, b1, W2, b2, Wu1, bu1, Wu2, bu2):
    pad = IDX_ROWS * LN - edge_index.shape[1]
    src2d = jnp.concatenate(
        [edge_index[0], jnp.zeros((pad,), jnp.int32)]).reshape(IDX_ROWS, LN)
    dst2d = jnp.concatenate(
        [edge_index[1], jnp.full((pad,), DUMMY, jnp.int32)]).reshape(IDX_ROWS, LN)
    ones16 = jnp.ones((LN, 16), jnp.float32)
    zeros16 = jnp.zeros((640, 16), jnp.float32)
    zeros128 = jnp.zeros((640, 128), jnp.float32)

    deg16 = _deg_kernel(dst2d, ones16, zeros16)[:, :N_NODES]

    t1 = pl.pallas_call(
        _t1_body,
        grid=(GRID,),
        in_specs=[
            _rows(), _rows((NC, R, 16)), _full((128, 256)), _full((128, 256)),
            _full((1, 256)), _full((256, 128)), _full((1, 128)),
        ],
        out_specs=[_rows((2, R, 128)), _rows(), _full((1, 128))],
        out_shape=[
            jax.ShapeDtypeStruct((2, N_NODES, 128), jnp.float32),
            jax.ShapeDtypeStruct((N_NODES, 128), jnp.float32),
            jax.ShapeDtypeStruct((1, 128), jnp.float32),
        ],
    )
    y1s, h_node, gsum = t1(x, deg16, W1, Wu1, bu1.reshape(1, 256), Wu2,
                           bu2.reshape(1, 128))

    acc1a = _edge_pass(y1s[0], src2d, dst2d, zeros128)
    acc1b = _edge_pass(y1s[1], src2d, dst2d, zeros128)

    t2 = pl.pallas_call(
        _t2_body,
        grid=(GRID,),
        in_specs=[
            _rows((2, R, 128)), _rows((NC, R, 128)), _rows((NC, R, 128)),
            _rows((NC, R, 16)), _full((1, 256)), _full((256, 128)),
        ],
        out_specs=_rows(),
        out_shape=jax.ShapeDtypeStruct((N_NODES, 128), jnp.float32),
    )
    y2 = t2(y1s, acc1a, acc1b, deg16, b1.reshape(1, 256), W2)

    acc2 = _edge_pass(y2, src2d, dst2d, zeros128)

    t3 = pl.pallas_call(
        _t3_body,
        grid=(GRID,),
        in_specs=[
            _rows(), _rows((NC, R, 128)), _rows((NC, R, 16)), _full((1, 128)),
        ],
        out_specs=_rows(),
        out_shape=jax.ShapeDtypeStruct((N_NODES, 128), jnp.float32),
    )
    h_gnn = t3(y2, acc2, deg16, b2.reshape(1, 128))

    h_graph = gsum[0] / N_NODES
    return (h_gnn, h_node, h_node, h_graph)


# speedup vs baseline: 1.0379x; 1.0379x over previous
import jax
import jax.numpy as jnp
from jax import lax
from jax.experimental import pallas as pl
from jax.experimental.pallas import tpu as pltpu
from jax.experimental.pallas import tpu_sc as plsc

N_NODES = 10000
D_IN = 128
HID = 256
OUT = 128

NC = 2
NS = 16
LN = 128
DUMMY = N_NODES
ACC_ROWS = 10240
ZCH = ACC_ROWS // NS // LN
NGT0 = 96
NGT1 = 64
NGTP = NGT0 + NGT1
IDX_ROWS = NS * NGTP + 16
EDGES_PAD = NS * NGTP * LN
NGH0 = NGT0 // 2


def _sc_mesh():
    return plsc.VectorSubcoreMesh(
        core_axis_name="c", subcore_axis_name="s", num_cores=NC, num_subcores=NS)


def _deg_body(dst_hbm, ones_hbm, zeros_hbm, out_hbm, didx, ones_v, acc,
              ssem):
    c = lax.axis_index("c")
    s = lax.axis_index("s")
    pltpu.sync_copy(zeros_hbm, acc.at[pl.ds(s * 640, 640)])
    pltpu.sync_copy(ones_hbm, ones_v)
    row0 = s * NGTP + c * NGT0
    pltpu.sync_copy(dst_hbm.at[pl.ds(row0, NGT0)], didx)
    plsc.subcore_barrier()
    K = 8

    @pl.loop(0, NGT0 // K - c * ((NGT0 - NGT1) // K))
    def _(g):
        cps = [
            pltpu.async_copy(ones_v, acc.at[didx.at[g * K + j]], ssem, add=True)
            for j in range(K)
        ]
        for cp in cps:
            cp.wait()

    plsc.subcore_barrier()
    pltpu.sync_copy(acc.at[pl.ds(s * 640, 640)],
                    out_hbm.at[c, pl.ds(s * 640, 640)])


def _deg_kernel(dst2d, ones16, zeros16):
    k = pl.kernel(
        _deg_body,
        out_type=jax.ShapeDtypeStruct((NC, ACC_ROWS, 16), jnp.float32),
        mesh=_sc_mesh(),
        compiler_params=pltpu.CompilerParams(use_tc_tiling_on_sc=False),
        scratch_types=[
            pltpu.VMEM((NGT0, LN), jnp.int32),
            pltpu.VMEM((LN, 16), jnp.float32),
            pltpu.VMEM_SHARED((ACC_ROWS, 16), jnp.float32),
            pltpu.SemaphoreType.DMA,
        ],
    )
    return k(dst2d, ones16, zeros16)


def _edge_body(y_hbm, src_hbm, dst_hbm, zeros_hbm, out_hbm,
               sidx, didx, buf0, buf1, acc, gsem, ssem):
    c = lax.axis_index("c")
    s = lax.axis_index("s")
    pltpu.sync_copy(zeros_hbm, buf0)
    for i in range(ZCH):
        pltpu.sync_copy(buf0, acc.at[pl.ds(s * 640 + i * LN, LN)])
    row0 = s * NGTP + c * NGT0
    ngh = NGH0 - c * ((NGT0 - NGT1) // 2)
    pairs = ngh // 2
    pltpu.sync_copy(src_hbm.at[pl.ds(row0, NGH0)], sidx)
    pltpu.sync_copy(dst_hbm.at[pl.ds(row0, NGH0)], didx)
    plsc.subcore_barrier()

    def fire_gather(k, buf):
        pltpu.async_copy(y_hbm.at[sidx.at[k]], buf, gsem)

    def fire_scatter(k, buf):
        pltpu.async_copy(buf, acc.at[didx.at[k]], ssem, add=True)

    def drain(sem, buf):
        pltpu.make_async_copy(y_hbm.at[pl.ds(0, LN)], buf, sem).wait()

    for h in range(2):
        if h:
            pltpu.sync_copy(src_hbm.at[pl.ds(row0 + ngh, NGH0)], sidx)
            pltpu.sync_copy(dst_hbm.at[pl.ds(row0 + ngh, NGH0)], didx)
        fire_gather(0, buf0)

        @pl.loop(0, pairs)
        def _(t):
            k0 = 2 * t
            k1 = k0 + 1
            fire_gather(k1, buf1)
            drain(gsem, buf0)
            fire_scatter(k0, buf0)
            drain(ssem, buf0)

            @pl.when(t < pairs - 1)
            def _():
                fire_gather(k0 + 2, buf0)

            drain(gsem, buf1)
            fire_scatter(k1, buf1)
            drain(ssem, buf1)

    plsc.subcore_barrier()
    pltpu.sync_copy(acc.at[pl.ds(s * 640, 640)],
                    out_hbm.at[c, pl.ds(s * 640, 640)])


def _edge_pass(y, src2d, dst2d, zeros128):
    k = pl.kernel(
        _edge_body,
        out_type=jax.ShapeDtypeStruct((NC, ACC_ROWS, 128), jnp.float32),
        mesh=_sc_mesh(),
        compiler_params=pltpu.CompilerParams(use_tc_tiling_on_sc=False),
        scratch_types=[
            pltpu.VMEM((NGH0, LN), jnp.int32),
            pltpu.VMEM((NGH0, LN), jnp.int32),
            pltpu.VMEM((LN, 128), jnp.float32),
            pltpu.VMEM((LN, 128), jnp.float32),
            pltpu.VMEM_SHARED((ACC_ROWS, 128), jnp.float32),
            pltpu.SemaphoreType.DMA,
            pltpu.SemaphoreType.DMA,
        ],
    )
    return k(y, src2d, dst2d, zeros128)[:, :N_NODES]


R = 1000
GRID = N_NODES // R


def _dinv_of(deg_ref):
    deg = deg_ref[0, :, 0:1] + deg_ref[1, :, 0:1] + 1.0
    return lax.rsqrt(deg)


def _t1_body(x_ref, deg_ref, w1_ref, wu1_ref, bu1_ref, wu2_ref, bu2_ref,
             y1_ref, hnode_ref, gsum_ref):
    i = pl.program_id(0)
    x = x_ref[...]
    dinv = _dinv_of(deg_ref)
    xw = jnp.dot(x, w1_ref[...], preferred_element_type=jnp.float32)
    y1 = xw * dinv
    y1_ref[0] = y1[:, :128]
    y1_ref[1] = y1[:, 128:]
    hu = jnp.maximum(
        jnp.dot(x, wu1_ref[...], preferred_element_type=jnp.float32) + bu1_ref[...],
        0.0)
    hnode = jnp.dot(hu, wu2_ref[...], preferred_element_type=jnp.float32) + bu2_ref[...]
    hnode_ref[...] = hnode
    part = jnp.sum(hnode, axis=0, keepdims=True)

    @pl.when(i == 0)
    def _():
        gsum_ref[...] = part

    @pl.when(i != 0)
    def _():
        gsum_ref[...] += part


def _t2_body(y1_ref, a1a_ref, a1b_ref, deg_ref, b1_ref, w2_ref, y2_ref):
    dinv = _dinv_of(deg_ref)
    ha = a1a_ref[0] + a1a_ref[1] + y1_ref[0]
    hb = a1b_ref[0] + a1b_ref[1] + y1_ref[1]
    h = jnp.concatenate([ha, hb], axis=1) * dinv + b1_ref[...]
    h = jnp.maximum(h, 0.0)
    y2_ref[...] = jnp.dot(h, w2_ref[...], preferred_element_type=jnp.float32) * dinv


def _t3_body(y2_ref, a2_ref, deg_ref, b2_ref, out_ref):
    dinv = _dinv_of(deg_ref)
    out = (a2_ref[0] + a2_ref[1] + y2_ref[...]) * dinv + b2_ref[...]
    out_ref[...] = jnp.maximum(out, 0.0)


def _full(shape):
    return pl.BlockSpec(shape, lambda i: tuple(0 for _ in shape))


def _rows(shape3=None):
    if shape3 is None:
        return pl.BlockSpec((R, 128), lambda i: (i, 0))
    return pl.BlockSpec(shape3, lambda i: (0, i, 0))


def kernel(x, edge_index, W1, b1, W2, b2, Wu1, bu1, Wu2, bu2):
    pad = IDX_ROWS * LN - edge_index.shape[1]
    src2d = jnp.concatenate(
        [edge_index[0], jnp.zeros((pad,), jnp.int32)]).reshape(IDX_ROWS, LN)
    dst2d = jnp.concatenate(
        [edge_index[1], jnp.full((pad,), DUMMY, jnp.int32)]).reshape(IDX_ROWS, LN)
    ones16 = jnp.ones((LN, 16), jnp.float32)
    zeros16 = jnp.zeros((640, 16), jnp.float32)
    zeros128 = jnp.zeros((LN, 128), jnp.float32)

    deg16 = _deg_kernel(dst2d, ones16, zeros16)[:, :N_NODES]

    t1 = pl.pallas_call(
        _t1_body,
        grid=(GRID,),
        in_specs=[
            _rows(), _rows((NC, R, 16)), _full((128, 256)), _full((128, 256)),
            _full((1, 256)), _full((256, 128)), _full((1, 128)),
        ],
        out_specs=[_rows((2, R, 128)), _rows(), _full((1, 128))],
        out_shape=[
            jax.ShapeDtypeStruct((2, N_NODES, 128), jnp.float32),
            jax.ShapeDtypeStruct((N_NODES, 128), jnp.float32),
            jax.ShapeDtypeStruct((1, 128), jnp.float32),
        ],
    )
    y1s, h_node, gsum = t1(x, deg16, W1, Wu1, bu1.reshape(1, 256), Wu2,
                           bu2.reshape(1, 128))

    acc1a = _edge_pass(y1s[0], src2d, dst2d, zeros128)
    acc1b = _edge_pass(y1s[1], src2d, dst2d, zeros128)

    t2 = pl.pallas_call(
        _t2_body,
        grid=(GRID,),
        in_specs=[
            _rows((2, R, 128)), _rows((NC, R, 128)), _rows((NC, R, 128)),
            _rows((NC, R, 16)), _full((1, 256)), _full((256, 128)),
        ],
        out_specs=_rows(),
        out_shape=jax.ShapeDtypeStruct((N_NODES, 128), jnp.float32),
    )
    y2 = t2(y1s, acc1a, acc1b, deg16, b1.reshape(1, 256), W2)

    acc2 = _edge_pass(y2, src2d, dst2d, zeros128)

    t3 = pl.pallas_call(
        _t3_body,
        grid=(GRID,),
        in_specs=[
            _rows(), _rows((NC, R, 128)), _rows((NC, R, 16)), _full((1, 128)),
        ],
        out_specs=_rows(),
        out_shape=jax.ShapeDtypeStruct((N_NODES, 128), jnp.float32),
    )
    h_gnn = t3(y2, acc2, deg16, b2.reshape(1, 128))

    h_graph = gsum[0] / N_NODES
    return (h_gnn, h_node, h_node, h_graph)


# speedup vs baseline: 1.1107x; 1.0701x over previous
import jax
import jax.numpy as jnp
from jax import lax
from jax.experimental import pallas as pl
from jax.experimental.pallas import tpu as pltpu
from jax.experimental.pallas import tpu_sc as plsc

N_NODES = 10000
D_IN = 128
HID = 256
OUT = 128

NC = 2
NS = 16
LN = 128
DUMMY = N_NODES
ACC_ROWS = 10240
ZCH = ACC_ROWS // NS // LN
NGT0 = 104
NGT1 = 56
NGTP = NGT0 + NGT1
IDX_ROWS = NS * NGTP + 16
EDGES_PAD = NS * NGTP * LN
NGH0 = NGT0 // 2


def _sc_mesh():
    return plsc.VectorSubcoreMesh(
        core_axis_name="c", subcore_axis_name="s", num_cores=NC, num_subcores=NS)


def _deg_body(dst_hbm, ones_hbm, zeros_hbm, out_hbm, didx, ones_v, acc,
              ssem):
    c = lax.axis_index("c")
    s = lax.axis_index("s")
    pltpu.sync_copy(zeros_hbm, acc.at[pl.ds(s * 640, 640)])
    pltpu.sync_copy(ones_hbm, ones_v)
    row0 = s * NGTP + c * NGT0
    pltpu.sync_copy(dst_hbm.at[pl.ds(row0, NGT0)], didx)
    plsc.subcore_barrier()
    K = 8

    @pl.loop(0, NGT0 // K - c * ((NGT0 - NGT1) // K))
    def _(g):
        cps = [
            pltpu.async_copy(ones_v, acc.at[didx.at[g * K + j]], ssem, add=True)
            for j in range(K)
        ]
        for cp in cps:
            cp.wait()

    plsc.subcore_barrier()
    pltpu.sync_copy(acc.at[pl.ds(s * 640, 640)],
                    out_hbm.at[c, pl.ds(s * 640, 640)])


def _deg_kernel(dst2d, ones16, zeros16):
    k = pl.kernel(
        _deg_body,
        out_type=jax.ShapeDtypeStruct((NC, ACC_ROWS, 16), jnp.float32),
        mesh=_sc_mesh(),
        compiler_params=pltpu.CompilerParams(use_tc_tiling_on_sc=False),
        scratch_types=[
            pltpu.VMEM((NGT0, LN), jnp.int32),
            pltpu.VMEM((LN, 16), jnp.float32),
            pltpu.VMEM_SHARED((ACC_ROWS, 16), jnp.float32),
            pltpu.SemaphoreType.DMA,
        ],
    )
    return k(dst2d, ones16, zeros16)


def _edge_body(y_hbm, src_hbm, dst_hbm, zeros_hbm, out_hbm,
               sidx, didx, buf0, buf1, acc, gsem, ssem):
    c = lax.axis_index("c")
    s = lax.axis_index("s")
    pltpu.sync_copy(zeros_hbm, buf0)
    for i in range(ZCH):
        pltpu.sync_copy(buf0, acc.at[pl.ds(s * 640 + i * LN, LN)])
    row0 = s * NGTP + c * NGT0
    ngh = NGH0 - c * ((NGT0 - NGT1) // 2)
    pairs = ngh // 2
    pltpu.sync_copy(src_hbm.at[pl.ds(row0, NGH0)], sidx)
    pltpu.sync_copy(dst_hbm.at[pl.ds(row0, NGH0)], didx)
    plsc.subcore_barrier()

    def fire_gather(k, buf):
        pltpu.async_copy(y_hbm.at[sidx.at[k]], buf, gsem)

    def fire_scatter(k, buf):
        pltpu.async_copy(buf, acc.at[didx.at[k]], ssem, add=True)

    def drain(sem, buf):
        pltpu.make_async_copy(y_hbm.at[pl.ds(0, LN)], buf, sem).wait()

    for h in range(2):
        if h:
            pltpu.sync_copy(src_hbm.at[pl.ds(row0 + ngh, NGH0)], sidx)
            pltpu.sync_copy(dst_hbm.at[pl.ds(row0 + ngh, NGH0)], didx)
        fire_gather(0, buf0)

        @pl.loop(0, pairs)
        def _(t):
            k0 = 2 * t
            k1 = k0 + 1
            fire_gather(k1, buf1)
            drain(gsem, buf0)
            fire_scatter(k0, buf0)
            drain(ssem, buf0)

            @pl.when(t < pairs - 1)
            def _():
                fire_gather(k0 + 2, buf0)

            drain(gsem, buf1)
            fire_scatter(k1, buf1)
            drain(ssem, buf1)

    plsc.subcore_barrier()
    pltpu.sync_copy(acc.at[pl.ds(s * 640, 640)],
                    out_hbm.at[c, pl.ds(s * 640, 640)])


def _edge_pass(y, src2d, dst2d, zeros128):
    k = pl.kernel(
        _edge_body,
        out_type=jax.ShapeDtypeStruct((NC, ACC_ROWS, 128), jnp.float32),
        mesh=_sc_mesh(),
        compiler_params=pltpu.CompilerParams(use_tc_tiling_on_sc=False),
        scratch_types=[
            pltpu.VMEM((NGH0, LN), jnp.int32),
            pltpu.VMEM((NGH0, LN), jnp.int32),
            pltpu.VMEM((LN, 128), jnp.float32),
            pltpu.VMEM((LN, 128), jnp.float32),
            pltpu.VMEM_SHARED((ACC_ROWS, 128), jnp.float32),
            pltpu.SemaphoreType.DMA,
            pltpu.SemaphoreType.DMA,
        ],
    )
    return k(y, src2d, dst2d, zeros128)[:, :N_NODES]


R = 1000
GRID = N_NODES // R


def _dinv_of(deg_ref):
    deg = deg_ref[0, :, 0:1] + deg_ref[1, :, 0:1] + 1.0
    return lax.rsqrt(deg)


def _t1_body(x_ref, deg_ref, w1_ref, wu1_ref, bu1_ref, wu2_ref, bu2_ref,
             y1_ref, hnode_ref, gsum_ref):
    i = pl.program_id(0)
    x = x_ref[...]
    dinv = _dinv_of(deg_ref)
    xw = jnp.dot(x, w1_ref[...], preferred_element_type=jnp.float32)
    y1 = xw * dinv
    y1_ref[0] = y1[:, :128]
    y1_ref[1] = y1[:, 128:]
    hu = jnp.maximum(
        jnp.dot(x, wu1_ref[...], preferred_element_type=jnp.float32) + bu1_ref[...],
        0.0)
    hnode = jnp.dot(hu, wu2_ref[...], preferred_element_type=jnp.float32) + bu2_ref[...]
    hnode_ref[...] = hnode
    part = jnp.sum(hnode, axis=0, keepdims=True)

    @pl.when(i == 0)
    def _():
        gsum_ref[...] = part

    @pl.when(i != 0)
    def _():
        gsum_ref[...] += part


def _t2_body(y1_ref, a1a_ref, a1b_ref, deg_ref, b1_ref, w2_ref, y2_ref):
    dinv = _dinv_of(deg_ref)
    ha = a1a_ref[0] + a1a_ref[1] + y1_ref[0]
    hb = a1b_ref[0] + a1b_ref[1] + y1_ref[1]
    h = jnp.concatenate([ha, hb], axis=1) * dinv + b1_ref[...]
    h = jnp.maximum(h, 0.0)
    y2_ref[...] = jnp.dot(h, w2_ref[...], preferred_element_type=jnp.float32) * dinv


def _t3_body(y2_ref, a2_ref, deg_ref, b2_ref, out_ref):
    dinv = _dinv_of(deg_ref)
    out = (a2_ref[0] + a2_ref[1] + y2_ref[...]) * dinv + b2_ref[...]
    out_ref[...] = jnp.maximum(out, 0.0)


def _full(shape):
    return pl.BlockSpec(shape, lambda i: tuple(0 for _ in shape))


def _rows(shape3=None):
    if shape3 is None:
        return pl.BlockSpec((R, 128), lambda i: (i, 0))
    return pl.BlockSpec(shape3, lambda i: (0, i, 0))


def kernel(x, edge_index, W1, b1, W2, b2, Wu1, bu1, Wu2, bu2):
    pad = IDX_ROWS * LN - edge_index.shape[1]
    src2d = jnp.concatenate(
        [edge_index[0], jnp.zeros((pad,), jnp.int32)]).reshape(IDX_ROWS, LN)
    dst2d = jnp.concatenate(
        [edge_index[1], jnp.full((pad,), DUMMY, jnp.int32)]).reshape(IDX_ROWS, LN)
    ones16 = jnp.ones((LN, 16), jnp.float32)
    zeros16 = jnp.zeros((640, 16), jnp.float32)
    zeros128 = jnp.zeros((LN, 128), jnp.float32)

    deg16 = _deg_kernel(dst2d, ones16, zeros16)[:, :N_NODES]

    t1 = pl.pallas_call(
        _t1_body,
        grid=(GRID,),
        in_specs=[
            _rows(), _rows((NC, R, 16)), _full((128, 256)), _full((128, 256)),
            _full((1, 256)), _full((256, 128)), _full((1, 128)),
        ],
        out_specs=[_rows((2, R, 128)), _rows(), _full((1, 128))],
        out_shape=[
            jax.ShapeDtypeStruct((2, N_NODES, 128), jnp.float32),
            jax.ShapeDtypeStruct((N_NODES, 128), jnp.float32),
            jax.ShapeDtypeStruct((1, 128), jnp.float32),
        ],
    )
    y1s, h_node, gsum = t1(x, deg16, W1, Wu1, bu1.reshape(1, 256), Wu2,
                           bu2.reshape(1, 128))

    acc1a = _edge_pass(y1s[0], src2d, dst2d, zeros128)
    acc1b = _edge_pass(y1s[1], src2d, dst2d, zeros128)

    t2 = pl.pallas_call(
        _t2_body,
        grid=(GRID,),
        in_specs=[
            _rows((2, R, 128)), _rows((NC, R, 128)), _rows((NC, R, 128)),
            _rows((NC, R, 16)), _full((1, 256)), _full((256, 128)),
        ],
        out_specs=_rows(),
        out_shape=jax.ShapeDtypeStruct((N_NODES, 128), jnp.float32),
    )
    y2 = t2(y1s, acc1a, acc1b, deg16, b1.reshape(1, 256), W2)

    acc2 = _edge_pass(y2, src2d, dst2d, zeros128)

    t3 = pl.pallas_call(
        _t3_body,
        grid=(GRID,),
        in_specs=[
            _rows(), _rows((NC, R, 128)), _rows((NC, R, 16)), _full((1, 128)),
        ],
        out_specs=_rows(),
        out_shape=jax.ShapeDtypeStruct((N_NODES, 128), jnp.float32),
    )
    h_gnn = t3(y2, acc2, deg16, b2.reshape(1, 128))

    h_graph = gsum[0] / N_NODES
    return (h_gnn, h_node, h_node, h_graph)


# speedup vs baseline: 1.1683x; 1.0519x over previous
import jax
import jax.numpy as jnp
from jax import lax
from jax.experimental import pallas as pl
from jax.experimental.pallas import tpu as pltpu
from jax.experimental.pallas import tpu_sc as plsc

N_NODES = 10000
D_IN = 128
HID = 256
OUT = 128

NC = 2
NS = 16
LN = 128
DUMMY = N_NODES
ACC_ROWS = 10240
ZCH = ACC_ROWS // NS // LN
NGT0 = 112
NGT1 = 48
NGTP = NGT0 + NGT1
IDX_ROWS = NS * NGTP + 16
EDGES_PAD = NS * NGTP * LN
NGH0 = NGT0 // 2


def _sc_mesh():
    return plsc.VectorSubcoreMesh(
        core_axis_name="c", subcore_axis_name="s", num_cores=NC, num_subcores=NS)


def _deg_body(dst_hbm, ones_hbm, zeros_hbm, out_hbm, didx, ones_v, acc,
              ssem):
    c = lax.axis_index("c")
    s = lax.axis_index("s")
    pltpu.sync_copy(zeros_hbm, acc.at[pl.ds(s * 640, 640)])
    pltpu.sync_copy(ones_hbm, ones_v)
    row0 = s * NGTP + c * NGT0
    pltpu.sync_copy(dst_hbm.at[pl.ds(row0, NGT0)], didx)
    plsc.subcore_barrier()
    K = 8

    @pl.loop(0, NGT0 // K - c * ((NGT0 - NGT1) // K))
    def _(g):
        cps = [
            pltpu.async_copy(ones_v, acc.at[didx.at[g * K + j]], ssem, add=True)
            for j in range(K)
        ]
        for cp in cps:
            cp.wait()

    plsc.subcore_barrier()
    pltpu.sync_copy(acc.at[pl.ds(s * 640, 640)],
                    out_hbm.at[c, pl.ds(s * 640, 640)])


def _deg_kernel(dst2d, ones16, zeros16):
    k = pl.kernel(
        _deg_body,
        out_type=jax.ShapeDtypeStruct((NC, ACC_ROWS, 16), jnp.float32),
        mesh=_sc_mesh(),
        compiler_params=pltpu.CompilerParams(use_tc_tiling_on_sc=False),
        scratch_types=[
            pltpu.VMEM((NGT0, LN), jnp.int32),
            pltpu.VMEM((LN, 16), jnp.float32),
            pltpu.VMEM_SHARED((ACC_ROWS, 16), jnp.float32),
            pltpu.SemaphoreType.DMA,
        ],
    )
    return k(dst2d, ones16, zeros16)


def _edge_body(y_hbm, src_hbm, dst_hbm, zeros_hbm, out_hbm,
               sidx, didx, buf0, buf1, acc, gsem, ssem):
    c = lax.axis_index("c")
    s = lax.axis_index("s")
    pltpu.sync_copy(zeros_hbm, buf0)
    for i in range(ZCH):
        pltpu.sync_copy(buf0, acc.at[pl.ds(s * 640 + i * LN, LN)])
    row0 = s * NGTP + c * NGT0
    ngh = NGH0 - c * ((NGT0 - NGT1) // 2)
    pairs = ngh // 2
    pltpu.sync_copy(src_hbm.at[pl.ds(row0, NGH0)], sidx)
    pltpu.sync_copy(dst_hbm.at[pl.ds(row0, NGH0)], didx)
    plsc.subcore_barrier()

    def fire_gather(k, buf):
        pltpu.async_copy(y_hbm.at[sidx.at[k]], buf, gsem)

    def fire_scatter(k, buf):
        pltpu.async_copy(buf, acc.at[didx.at[k]], ssem, add=True)

    def drain(sem, buf):
        pltpu.make_async_copy(y_hbm.at[pl.ds(0, LN)], buf, sem).wait()

    for h in range(2):
        if h:
            pltpu.sync_copy(src_hbm.at[pl.ds(row0 + ngh, NGH0)], sidx)
            pltpu.sync_copy(dst_hbm.at[pl.ds(row0 + ngh, NGH0)], didx)
        fire_gather(0, buf0)

        @pl.loop(0, pairs)
        def _(t):
            k0 = 2 * t
            k1 = k0 + 1
            fire_gather(k1, buf1)
            drain(gsem, buf0)
            fire_scatter(k0, buf0)
            drain(ssem, buf0)

            @pl.when(t < pairs - 1)
            def _():
                fire_gather(k0 + 2, buf0)

            drain(gsem, buf1)
            fire_scatter(k1, buf1)
            drain(ssem, buf1)

    plsc.subcore_barrier()
    pltpu.sync_copy(acc.at[pl.ds(s * 640, 640)],
                    out_hbm.at[c, pl.ds(s * 640, 640)])


def _edge_pass(y, src2d, dst2d, zeros128):
    k = pl.kernel(
        _edge_body,
        out_type=jax.ShapeDtypeStruct((NC, ACC_ROWS, 128), jnp.float32),
        mesh=_sc_mesh(),
        compiler_params=pltpu.CompilerParams(use_tc_tiling_on_sc=False),
        scratch_types=[
            pltpu.VMEM((NGH0, LN), jnp.int32),
            pltpu.VMEM((NGH0, LN), jnp.int32),
            pltpu.VMEM((LN, 128), jnp.float32),
            pltpu.VMEM((LN, 128), jnp.float32),
            pltpu.VMEM_SHARED((ACC_ROWS, 128), jnp.float32),
            pltpu.SemaphoreType.DMA,
            pltpu.SemaphoreType.DMA,
        ],
    )
    return k(y, src2d, dst2d, zeros128)[:, :N_NODES]


R = 1000
GRID = N_NODES // R


def _dinv_of(deg_ref):
    deg = deg_ref[0, :, 0:1] + deg_ref[1, :, 0:1] + 1.0
    return lax.rsqrt(deg)


def _t1_body(x_ref, deg_ref, w1_ref, wu1_ref, bu1_ref, wu2_ref, bu2_ref,
             y1_ref, hnode_ref, gsum_ref):
    i = pl.program_id(0)
    x = x_ref[...]
    dinv = _dinv_of(deg_ref)
    xw = jnp.dot(x, w1_ref[...], preferred_element_type=jnp.float32)
    y1 = xw * dinv
    y1_ref[0] = y1[:, :128]
    y1_ref[1] = y1[:, 128:]
    hu = jnp.maximum(
        jnp.dot(x, wu1_ref[...], preferred_element_type=jnp.float32) + bu1_ref[...],
        0.0)
    hnode = jnp.dot(hu, wu2_ref[...], preferred_element_type=jnp.float32) + bu2_ref[...]
    hnode_ref[...] = hnode
    part = jnp.sum(hnode, axis=0, keepdims=True)

    @pl.when(i == 0)
    def _():
        gsum_ref[...] = part

    @pl.when(i != 0)
    def _():
        gsum_ref[...] += part


def _t2_body(y1_ref, a1a_ref, a1b_ref, deg_ref, b1_ref, w2_ref, y2_ref):
    dinv = _dinv_of(deg_ref)
    ha = a1a_ref[0] + a1a_ref[1] + y1_ref[0]
    hb = a1b_ref[0] + a1b_ref[1] + y1_ref[1]
    h = jnp.concatenate([ha, hb], axis=1) * dinv + b1_ref[...]
    h = jnp.maximum(h, 0.0)
    y2_ref[...] = jnp.dot(h, w2_ref[...], preferred_element_type=jnp.float32) * dinv


def _t3_body(y2_ref, a2_ref, deg_ref, b2_ref, out_ref):
    dinv = _dinv_of(deg_ref)
    out = (a2_ref[0] + a2_ref[1] + y2_ref[...]) * dinv + b2_ref[...]
    out_ref[...] = jnp.maximum(out, 0.0)


def _full(shape):
    return pl.BlockSpec(shape, lambda i: tuple(0 for _ in shape))


def _rows(shape3=None):
    if shape3 is None:
        return pl.BlockSpec((R, 128), lambda i: (i, 0))
    return pl.BlockSpec(shape3, lambda i: (0, i, 0))


def kernel(x, edge_index, W1, b1, W2, b2, Wu1, bu1, Wu2, bu2):
    pad = IDX_ROWS * LN - edge_index.shape[1]
    src2d = jnp.concatenate(
        [edge_index[0], jnp.zeros((pad,), jnp.int32)]).reshape(IDX_ROWS, LN)
    dst2d = jnp.concatenate(
        [edge_index[1], jnp.full((pad,), DUMMY, jnp.int32)]).reshape(IDX_ROWS, LN)
    ones16 = jnp.ones((LN, 16), jnp.float32)
    zeros16 = jnp.zeros((640, 16), jnp.float32)
    zeros128 = jnp.zeros((LN, 128), jnp.float32)

    deg16 = _deg_kernel(dst2d, ones16, zeros16)[:, :N_NODES]

    t1 = pl.pallas_call(
        _t1_body,
        grid=(GRID,),
        in_specs=[
            _rows(), _rows((NC, R, 16)), _full((128, 256)), _full((128, 256)),
            _full((1, 256)), _full((256, 128)), _full((1, 128)),
        ],
        out_specs=[_rows((2, R, 128)), _rows(), _full((1, 128))],
        out_shape=[
            jax.ShapeDtypeStruct((2, N_NODES, 128), jnp.float32),
            jax.ShapeDtypeStruct((N_NODES, 128), jnp.float32),
            jax.ShapeDtypeStruct((1, 128), jnp.float32),
        ],
    )
    y1s, h_node, gsum = t1(x, deg16, W1, Wu1, bu1.reshape(1, 256), Wu2,
                           bu2.reshape(1, 128))

    acc1a = _edge_pass(y1s[0], src2d, dst2d, zeros128)
    acc1b = _edge_pass(y1s[1], src2d, dst2d, zeros128)

    t2 = pl.pallas_call(
        _t2_body,
        grid=(GRID,),
        in_specs=[
            _rows((2, R, 128)), _rows((NC, R, 128)), _rows((NC, R, 128)),
            _rows((NC, R, 16)), _full((1, 256)), _full((256, 128)),
        ],
        out_specs=_rows(),
        out_shape=jax.ShapeDtypeStruct((N_NODES, 128), jnp.float32),
    )
    y2 = t2(y1s, acc1a, acc1b, deg16, b1.reshape(1, 256), W2)

    acc2 = _edge_pass(y2, src2d, dst2d, zeros128)

    t3 = pl.pallas_call(
        _t3_body,
        grid=(GRID,),
        in_specs=[
            _rows(), _rows((NC, R, 128)), _rows((NC, R, 16)), _full((1, 128)),
        ],
        out_specs=_rows(),
        out_shape=jax.ShapeDtypeStruct((N_NODES, 128), jnp.float32),
    )
    h_gnn = t3(y2, acc2, deg16, b2.reshape(1, 128))

    h_graph = gsum[0] / N_NODES
    return (h_gnn, h_node, h_node, h_graph)
